# baseline (speedup 1.0000x reference)
import functools

import jax
import jax.numpy as jnp
from jax import lax
from jax.experimental import pallas as pl
from jax.experimental.pallas import tpu as pltpu
from jax.experimental.pallas import tpu_sc as plsc

_B = 16384
_H = 64
_NC = 2
_NS = 16
_NW = _NC * _NS
_W = _B // _NW
_C = 16
_NCHUNK = _W // _C
_L = 16
_MARGIN = 1.0
_EPS = 1e-12


def _rsqrt(x):
    i = plsc.bitcast(x, jnp.int32)
    y = plsc.bitcast(jnp.int32(0x5F3759DF) - (i >> 1), jnp.float32)
    for _ in range(3):
        y = y * (1.5 - 0.5 * x * y * y)
    return y


def _transh_body(ph_hbm, pt_hbm, pr_hbm, nh_hbm, nt_hbm, nr_hbm,
                 ent_hbm, rel_hbm, nrm_hbm, out_hbm,
                 iph, ipt, ipr, inh, int_, inr,
                 stgh, stgt, stgr, stgn,
                 acc, sem):
    cid = lax.axis_index("c")
    sid = lax.axis_index("s")
    wid = sid * _NC + cid
    base = wid * _W
    zero = jnp.zeros((_L,), jnp.float32)
    iota16 = lax.iota(jnp.int32, _L)
    acc[...] = zero

    def side(vh, vt, vr):
        t8h = (vh >> 3) << 3
        t8t = (vt >> 3) << 3
        t8r = (vr >> 3) << 3
        d = []
        for k in range(_L):
            dst8 = pl.ds(k * 8, 8)
            for tv, tab, stg in ((t8h, ent_hbm, stgh), (t8t, ent_hbm, stgt),
                                 (t8r, rel_hbm, stgr), (t8r, nrm_hbm, stgn)):
                t8 = pl.multiple_of(tv[k], 8)
                d.append(pltpu.async_copy(tab.at[pl.ds(t8, 8)],
                                          stg.at[dst8], sem))
        for dd in d:
            dd.wait()

        rows8 = iota16 * 8
        rh_ = rows8 + (vh & 7)
        rt_ = rows8 + (vt & 7)
        rr_ = rows8 + (vr & 7)

        def p1(j, s):
            nn, hn, tn, hh, tt, rr = s
            j4 = j * 4
            colb = jnp.full((_L,), j4, jnp.int32)
            ld = []
            for q in range(4):
                col = colb + q
                h = plsc.load_gather(stgh, [rh_, col])
                t = plsc.load_gather(stgt, [rt_, col])
                r = plsc.load_gather(stgr, [rr_, col])
                n = plsc.load_gather(stgn, [rr_, col])
                ld.append((h, t, r, n))
            for pair in ((0, 1), (2, 3)):
                (h0, t0, r0, n0), (h1, t1, r1, n1) = ld[pair[0]], ld[pair[1]]
                nn = nn + (n0 * n0 + n1 * n1)
                hn = hn + (h0 * n0 + h1 * n1)
                tn = tn + (t0 * n0 + t1 * n1)
                hh = hh + (h0 * h0 + h1 * h1)
                tt = tt + (t0 * t0 + t1 * t1)
                rr = rr + (r0 * r0 + r1 * r1)
            return (nn, hn, tn, hh, tt, rr)

        nn, hn, tn, hh, tt, rr = lax.fori_loop(0, _H // 4, p1, (zero,) * 6)
        nnc = jnp.maximum(nn, _EPS)
        a = hn / nnc
        b = tn / nnc
        phsq = hh - 2.0 * a * hn + a * a * nn
        ptsq = tt - 2.0 * b * tn + b * b * nn
        rhv = _rsqrt(jnp.maximum(phsq, _EPS))
        rtv = _rsqrt(jnp.maximum(ptsq, _EPS))
        rrv = _rsqrt(jnp.maximum(rr, _EPS))
        dv = rhv * a - rtv * b

        def p2(j, s):
            j4 = j * 4
            colb = jnp.full((_L,), j4, jnp.int32)
            terms = []
            for q in range(4):
                col = colb + q
                h = plsc.load_gather(stgh, [rh_, col])
                t = plsc.load_gather(stgt, [rt_, col])
                r = plsc.load_gather(stgr, [rr_, col])
                n = plsc.load_gather(stgn, [rr_, col])
                terms.append(jnp.abs(rhv * h + rrv * r - rtv * t - dv * n))
            return s + ((terms[0] + terms[1]) + (terms[2] + terms[3]))

        return lax.fori_loop(0, _H // 4, p2, zero)

    d0 = [pltpu.async_copy(ph_hbm.at[pl.ds(base, _W)], iph, sem),
          pltpu.async_copy(pt_hbm.at[pl.ds(base, _W)], ipt, sem),
          pltpu.async_copy(pr_hbm.at[pl.ds(base, _W)], ipr, sem),
          pltpu.async_copy(nh_hbm.at[pl.ds(base, _W)], inh, sem),
          pltpu.async_copy(nt_hbm.at[pl.ds(base, _W)], int_, sem),
          pltpu.async_copy(nr_hbm.at[pl.ds(base, _W)], inr, sem)]
    for dd in d0:
        dd.wait()

    def chunk_body(c, carry):
        sl = pl.ds(c * _C, _C)
        sp = side(iph[sl], ipt[sl], ipr[sl])
        sn = side(inh[sl], int_[sl], inr[sl])
        acc[...] = acc[...] + jnp.maximum(sp - sn + _MARGIN, 0.0)
        return carry

    lax.fori_loop(0, _NCHUNK, chunk_body, 0)
    pltpu.sync_copy(acc, out_hbm.at[pl.ds(wid * _L, _L)])


_transh_sc = functools.partial(
    pl.kernel,
    out_type=jax.ShapeDtypeStruct((_NW * _L,), jnp.float32),
    mesh=plsc.VectorSubcoreMesh(core_axis_name="c", subcore_axis_name="s",
                                num_cores=_NC, num_subcores=_NS),
    scratch_types=[
        pltpu.VMEM((_W,), jnp.int32),
        pltpu.VMEM((_W,), jnp.int32),
        pltpu.VMEM((_W,), jnp.int32),
        pltpu.VMEM((_W,), jnp.int32),
        pltpu.VMEM((_W,), jnp.int32),
        pltpu.VMEM((_W,), jnp.int32),
        pltpu.VMEM((_C * 8, _H), jnp.float32),
        pltpu.VMEM((_C * 8, _H), jnp.float32),
        pltpu.VMEM((_C * 8, _H), jnp.float32),
        pltpu.VMEM((_C * 8, _H), jnp.float32),
        pltpu.VMEM((_L,), jnp.float32),
        pltpu.SemaphoreType.DMA,
    ],
    compiler_params=pltpu.CompilerParams(needs_layout_passes=False),
)(_transh_body)


def kernel(pos_h, pos_t, pos_r, neg_h, neg_t, neg_r,
           ent_embeddings, rel_embeddings, normal_vectors):
    parts = _transh_sc(pos_h, pos_t, pos_r, neg_h, neg_t, neg_r,
                       ent_embeddings, rel_embeddings, normal_vectors)
    return jnp.sum(parts) * (1.0 / _B)
